# Initial kernel scaffold; baseline (speedup 1.0000x reference)
import jax
import jax.numpy as jnp
from jax.experimental import pallas as pl


def kernel(ids, weight):
    raise NotImplementedError("write your pallas kernel here")


# speedup vs baseline: 4.2477x; 4.2477x over previous
import functools

import jax
import jax.numpy as jnp
from jax import lax
from jax.experimental import pallas as pl
from jax.experimental.pallas import tpu as pltpu
from jax.experimental.pallas import tpu_sc as plsc

DIM = 64
SCALE = 0.125
NC, NS = 2, 16
NW = NC * NS
CHUNK = 400


@functools.lru_cache(maxsize=None)
def _build(B: int):
  assert B % NW == 0
  b_per_w = B // NW
  assert b_per_w % (2 * CHUNK) == 0
  n_chunks = b_per_w // CHUNK
  mesh = plsc.VectorSubcoreMesh(core_axis_name="c", subcore_axis_name="s")

  @functools.partial(
      pl.kernel,
      out_type=jax.ShapeDtypeStruct((B, DIM), jnp.float32),
      mesh=mesh,
      compiler_params=pltpu.CompilerParams(use_tc_tiling_on_sc=False),
      scratch_types=[
          pltpu.VMEM((CHUNK,), jnp.int32),
          pltpu.VMEM((CHUNK,), jnp.int32),
          pltpu.VMEM((CHUNK, DIM), jnp.float32),
          pltpu.VMEM((CHUNK, DIM), jnp.float32),
          pltpu.VMEM((CHUNK, DIM), jnp.float32),
          pltpu.VMEM((CHUNK, DIM), jnp.float32),
          pltpu.SemaphoreType.DMA,
          pltpu.SemaphoreType.DMA,
          pltpu.SemaphoreType.DMA,
          pltpu.SemaphoreType.DMA,
          pltpu.SemaphoreType.DMA,
          pltpu.SemaphoreType.DMA,
      ],
  )
  def embed(ids_hbm, w_hbm, out_hbm, idx0, idx1, g0, g1, s0, s1,
            sem_i0, sem_i1, sem_g0, sem_g1, sem_o0, sem_o1):
    wid = lax.axis_index("s") * NC + lax.axis_index("c")
    base = wid * b_per_w

    slots = (
        (idx0, g0, s0, sem_i0, sem_g0, sem_o0),
        (idx1, g1, s1, sem_i1, sem_g1, sem_o1),
    )

    for b, (idxb, gb, sb, sem_ib, sem_gb, sem_ob) in enumerate(slots):
      pltpu.sync_copy(ids_hbm.at[pl.ds(base + b * CHUNK, CHUNK)], idxb)
      pltpu.async_copy(w_hbm.at[idxb], gb, sem_gb)

    @pl.loop(0, n_chunks // 2)
    def _(i):
      for b, (idxb, gb, sb, sem_ib, sem_gb, sem_ob) in enumerate(slots):
        c = i * 2 + b
        pltpu.make_async_copy(w_hbm.at[idxb], gb, sem_gb).wait()

        @pl.when(c < n_chunks - 2)
        def _():
          pltpu.async_copy(
              ids_hbm.at[pl.ds(base + (c + 2) * CHUNK, CHUNK)], idxb, sem_ib)

        @pl.when(c >= 2)
        def _():
          pltpu.make_async_copy(
              sb, out_hbm.at[pl.ds(base, CHUNK)], sem_ob).wait()

        @pl.loop(0, CHUNK)
        def _(r):
          for cc in range(DIM // 16):
            sb[r, pl.ds(cc * 16, 16)] = gb[r, pl.ds(cc * 16, 16)] * SCALE

        pltpu.async_copy(
            sb, out_hbm.at[pl.ds(base + c * CHUNK, CHUNK)], sem_ob)

        @pl.when(c < n_chunks - 2)
        def _():
          pltpu.make_async_copy(
              ids_hbm.at[pl.ds(base, CHUNK)], idxb, sem_ib).wait()
          pltpu.async_copy(w_hbm.at[idxb], gb, sem_gb)

    for b, (idxb, gb, sb, sem_ib, sem_gb, sem_ob) in enumerate(slots):
      pltpu.make_async_copy(sb, out_hbm.at[pl.ds(base, CHUNK)], sem_ob).wait()

  return embed


def kernel(ids, weight):
  bsz, seq = ids.shape
  B = bsz * seq
  ids_flat = ids.reshape(B).astype(jnp.int32)
  out = _build(B)(ids_flat, weight)
  return out.reshape(bsz, seq, DIM)
